# baseline (speedup 1.0000x reference)
import functools

import jax
import jax.numpy as jnp
from jax import lax
from jax.experimental import pallas as pl
from jax.experimental.pallas import tpu as pltpu
from jax.experimental.pallas import tpu_sc as plsc

N_TOTAL = 2097152
N_SC = 1572864
N_TC = N_TOTAL - N_SC
N_BINS_OUT = 10
NC = 2
NS = 16
L = 16
NW = NC * NS
PER_W = N_SC // NW
BLK = 8192
NBLK = PER_W // BLK
TILE = 128
NTILE = BLK // TILE
HBINS = 11
HWORDS = HBINS * L
NREP = TILE // L
HTOT = NREP * HWORDS

_mesh = plsc.VectorSubcoreMesh(core_axis_name="c", subcore_axis_name="s")


@functools.partial(
    pl.kernel,
    mesh=_mesh,
    out_type=(
        jax.ShapeDtypeStruct((NW, HWORDS), jnp.int32),
        jax.ShapeDtypeStruct((NW, HWORDS), jnp.int32),
        jax.ShapeDtypeStruct((NW, HWORDS), jnp.float32),
    ),
    scratch_types=[
        pltpu.VMEM((2 * BLK,), jnp.float32),
        pltpu.VMEM((2 * BLK,), jnp.float32),
        pltpu.VMEM((BLK,), jnp.int32),
        pltpu.VMEM((BLK,), jnp.int32),
        pltpu.VMEM((HTOT,), jnp.int32),
        pltpu.VMEM((HTOT,), jnp.int32),
        pltpu.VMEM((HTOT,), jnp.float32),
        pltpu.SemaphoreType.DMA,
        pltpu.SemaphoreType.DMA,
    ],
    compiler_params=pltpu.CompilerParams(
        needs_layout_passes=False, use_tc_tiling_on_sc=False),
)
def _ece_hist(lg_hbm, lb_hbm, cnt_out, lab_out, prd_out,
              lg_a, lg_b, lb_a, lb_b, cnt_v, lab_v, prd_v, sem_a, sem_b):
    wid = lax.axis_index("s") * NC + lax.axis_index("c")

    lane = lax.iota(jnp.int32, L)
    ones_i = jnp.ones((L,), jnp.int32)
    z_i = jnp.zeros((L,), jnp.int32)
    z_f = jnp.zeros((L,), jnp.float32)

    for b in range(NREP * HBINS):
        cnt_v[pl.ds(b * L, L)] = z_i
        lab_v[pl.ds(b * L, L)] = z_i
        prd_v[pl.ds(b * L, L)] = z_f

    elem0 = wid * PER_W

    def start_blk(blk, lgbuf, lbbuf, sem):
        off = elem0 + blk * BLK
        pltpu.async_copy(lg_hbm.at[pl.ds(off * 2, 2 * BLK)], lgbuf, sem)
        pltpu.async_copy(lb_hbm.at[pl.ds(off, BLK)], lbbuf, sem)

    def wait_blk(lgbuf, lbbuf, sem):
        pltpu.make_async_copy(lg_hbm.at[pl.ds(0, 2 * BLK)], lgbuf, sem).wait()
        pltpu.make_async_copy(lb_hbm.at[pl.ds(0, BLK)], lbbuf, sem).wait()

    def compute(lg_v, lb_v):
        def body(t):
            base = t * (2 * TILE)
            lbase = t * TILE
            for i in range(TILE // L):
                l0 = lg_v[pl.ds(base + i * L, L)]
                l1 = lg_v[pl.ds(base + TILE + i * L, L)]
                lb16 = lb_v[pl.ds(lbase + i * L, L)]
                e = jnp.exp(l0 - l1)
                p = 1.0 / (1.0 + e)
                bin_ = (p * 10.0).astype(jnp.int32)
                addr = bin_ * L + lane
                rep = pl.ds(i * HWORDS, HWORDS)
                plsc.addupdate_scatter(cnt_v.at[rep], [addr], ones_i)
                plsc.addupdate_scatter(lab_v.at[rep], [addr], lb16)
                plsc.addupdate_scatter(prd_v.at[rep], [addr], p)

        plsc.parallel_loop(0, NTILE)(body)

    start_blk(0, lg_a, lb_a, sem_a)

    def super_body(k, c):
        blk_a = 2 * k
        wait_blk(lg_a, lb_a, sem_a)
        start_blk(blk_a + 1, lg_b, lb_b, sem_b)
        compute(lg_a, lb_a)
        wait_blk(lg_b, lb_b, sem_b)

        @pl.when(k < NBLK // 2 - 1)
        def _():
            start_blk(blk_a + 2, lg_a, lb_a, sem_a)

        compute(lg_b, lb_b)
        return c

    lax.fori_loop(0, NBLK // 2, super_body, 0)

    for b in range(HBINS):
        ci = cnt_v[pl.ds(b * L, L)]
        li = lab_v[pl.ds(b * L, L)]
        pi = prd_v[pl.ds(b * L, L)]
        for r in range(1, NREP):
            ci = ci + cnt_v[pl.ds(r * HWORDS + b * L, L)]
            li = li + lab_v[pl.ds(r * HWORDS + b * L, L)]
            pi = pi + prd_v[pl.ds(r * HWORDS + b * L, L)]
        cnt_v[pl.ds(b * L, L)] = ci
        lab_v[pl.ds(b * L, L)] = li
        prd_v[pl.ds(b * L, L)] = pi

    pltpu.sync_copy(cnt_v.at[pl.ds(0, HWORDS)], cnt_out.at[wid])
    pltpu.sync_copy(lab_v.at[pl.ds(0, HWORDS)], lab_out.at[wid])
    pltpu.sync_copy(prd_v.at[pl.ds(0, HWORDS)], prd_out.at[wid])


TC_RB = 512
TC_G = (N_TC // TILE) // TC_RB
_LG_ROW0 = (N_SC * 2) // TILE // (2 * TC_RB)
_LB_ROW0 = (N_SC // TILE) // TC_RB


def _tc_body(lg_ref, lb_ref, out_ref, acc_ref):
    g = pl.program_id(0)
    x = lg_ref[...].reshape(TC_RB, 2, 128)
    l0 = x[:, 0, :]
    l1 = x[:, 1, :]
    p = 1.0 / (1.0 + jnp.exp(l0 - l1))
    bins = (p * 10.0).astype(jnp.int32)
    lbf = lb_ref[...].astype(jnp.float32)
    ones = jnp.ones_like(p)
    zers = jnp.zeros_like(p)

    @pl.when(g == 0)
    def _():
        acc_ref[...] = jnp.zeros_like(acc_ref)

    for b in range(HBINS):
        m = bins == b
        acc_ref[b] += jnp.where(m, ones, zers)
        acc_ref[HBINS + b] += jnp.where(m, lbf, zers)
        acc_ref[2 * HBINS + b] += jnp.where(m, p, zers)

    @pl.when(g == TC_G - 1)
    def _():
        out_ref[...] = acc_ref[...].reshape(
            3 * HBINS, TC_RB // 8, 8, 128).sum(axis=1)


_tc_hist = pl.pallas_call(
    _tc_body,
    grid=(TC_G,),
    in_specs=[
        pl.BlockSpec((2 * TC_RB, 128), lambda g: (_LG_ROW0 + g, 0)),
        pl.BlockSpec((TC_RB, 128), lambda g: (_LB_ROW0 + g, 0)),
    ],
    out_specs=pl.BlockSpec((3 * HBINS, 8, 128), lambda g: (0, 0, 0)),
    out_shape=jax.ShapeDtypeStruct((3 * HBINS, 8, 128), jnp.float32),
    scratch_shapes=[pltpu.VMEM((3 * HBINS, TC_RB, 128), jnp.float32)],
)


def kernel(logits, labels):
    lg_flat = (logits.reshape(N_TOTAL // TILE, TILE, 2)
               .transpose(0, 2, 1).reshape(-1))
    cnt, lab, prd = _ece_hist(lg_flat, labels)
    tc = _tc_hist(lg_flat.reshape(N_TOTAL * 2 // TILE, TILE),
                  labels.reshape(N_TOTAL // TILE, TILE))
    tc_sums = tc.sum(axis=(1, 2))
    sizes = cnt.reshape(NW, HBINS, L).sum(axis=(0, 2)).astype(jnp.float32)
    lab_s = lab.reshape(NW, HBINS, L).sum(axis=(0, 2)).astype(jnp.float32)
    prd_s = prd.reshape(NW, HBINS, L).sum(axis=(0, 2))
    sizes = (sizes + tc_sums[:HBINS])[:N_BINS_OUT]
    lab_s = (lab_s + tc_sums[HBINS:2 * HBINS])[:N_BINS_OUT]
    prd_s = (prd_s + tc_sums[2 * HBINS:])[:N_BINS_OUT]
    nonempty = sizes > 0
    safe = jnp.where(nonempty, sizes, 1.0)
    accs = jnp.where(nonempty, lab_s / safe, 0.0)
    confs = jnp.where(nonempty, prd_s / safe, 0.0)
    return jnp.sum(sizes / jnp.sum(sizes) * jnp.abs(accs - confs))


# speedup vs baseline: 2.2716x; 2.2716x over previous
import functools

import jax
import jax.numpy as jnp
from jax import lax
from jax.experimental import pallas as pl
from jax.experimental.pallas import tpu as pltpu
from jax.experimental.pallas import tpu_sc as plsc

N_TOTAL = 2097152
N_SC = 1572864
N_TC = N_TOTAL - N_SC
N_BINS_OUT = 10
NC = 2
NS = 16
L = 16
NW = NC * NS
PER_W = N_SC // NW
BLK = 8192
NBLK = PER_W // BLK
TILE = 128
NTILE = BLK // TILE
HBINS = 11
HWORDS = HBINS * L
NREP = TILE // L
HTOT = NREP * HWORDS

_mesh = plsc.VectorSubcoreMesh(core_axis_name="c", subcore_axis_name="s")


@functools.partial(
    pl.kernel,
    mesh=_mesh,
    out_type=(
        jax.ShapeDtypeStruct((NW, HWORDS), jnp.int32),
        jax.ShapeDtypeStruct((NW, HWORDS), jnp.int32),
        jax.ShapeDtypeStruct((NW, HWORDS), jnp.float32),
    ),
    scratch_types=[
        pltpu.VMEM((2 * BLK,), jnp.float32),
        pltpu.VMEM((2 * BLK,), jnp.float32),
        pltpu.VMEM((BLK,), jnp.int32),
        pltpu.VMEM((BLK,), jnp.int32),
        pltpu.VMEM((HTOT,), jnp.int32),
        pltpu.VMEM((HTOT,), jnp.int32),
        pltpu.VMEM((HTOT,), jnp.float32),
        pltpu.SemaphoreType.DMA,
        pltpu.SemaphoreType.DMA,
    ],
    compiler_params=pltpu.CompilerParams(
        needs_layout_passes=False, use_tc_tiling_on_sc=False),
)
def _ece_hist(lg_hbm, lb_hbm, cnt_out, lab_out, prd_out,
              lg_a, lg_b, lb_a, lb_b, cnt_v, lab_v, prd_v, sem_a, sem_b):
    wid = lax.axis_index("s") * NC + lax.axis_index("c")

    lane = lax.iota(jnp.int32, L)
    ones_i = jnp.ones((L,), jnp.int32)
    z_i = jnp.zeros((L,), jnp.int32)
    z_f = jnp.zeros((L,), jnp.float32)

    for b in range(NREP * HBINS):
        cnt_v[pl.ds(b * L, L)] = z_i
        lab_v[pl.ds(b * L, L)] = z_i
        prd_v[pl.ds(b * L, L)] = z_f

    elem0 = wid * PER_W

    def start_blk(blk, lgbuf, lbbuf, sem):
        off = elem0 + blk * BLK
        pltpu.async_copy(lg_hbm.at[pl.ds(off * 2, 2 * BLK)], lgbuf, sem)
        pltpu.async_copy(lb_hbm.at[pl.ds(off, BLK)], lbbuf, sem)

    def wait_blk(lgbuf, lbbuf, sem):
        pltpu.make_async_copy(lg_hbm.at[pl.ds(0, 2 * BLK)], lgbuf, sem).wait()
        pltpu.make_async_copy(lb_hbm.at[pl.ds(0, BLK)], lbbuf, sem).wait()

    def compute(lg_v, lb_v):
        def body(t):
            base = t * (2 * TILE)
            lbase = t * TILE
            for i in range(TILE // L):
                l0 = lg_v[pl.ds(base + i * L, L)]
                l1 = lg_v[pl.ds(base + TILE + i * L, L)]
                lb16 = lb_v[pl.ds(lbase + i * L, L)]
                e = jnp.exp(l0 - l1)
                p = 1.0 / (1.0 + e)
                bin_ = (p * 10.0).astype(jnp.int32)
                addr = bin_ * L + lane
                rep = pl.ds(i * HWORDS, HWORDS)
                plsc.addupdate_scatter(cnt_v.at[rep], [addr], ones_i)
                plsc.addupdate_scatter(lab_v.at[rep], [addr], lb16)
                plsc.addupdate_scatter(prd_v.at[rep], [addr], p)

        plsc.parallel_loop(0, NTILE)(body)

    start_blk(0, lg_a, lb_a, sem_a)

    def super_body(k, c):
        blk_a = 2 * k
        wait_blk(lg_a, lb_a, sem_a)
        start_blk(blk_a + 1, lg_b, lb_b, sem_b)
        compute(lg_a, lb_a)
        wait_blk(lg_b, lb_b, sem_b)

        @pl.when(k < NBLK // 2 - 1)
        def _():
            start_blk(blk_a + 2, lg_a, lb_a, sem_a)

        compute(lg_b, lb_b)
        return c

    lax.fori_loop(0, NBLK // 2, super_body, 0)

    for b in range(HBINS):
        ci = cnt_v[pl.ds(b * L, L)]
        li = lab_v[pl.ds(b * L, L)]
        pi = prd_v[pl.ds(b * L, L)]
        for r in range(1, NREP):
            ci = ci + cnt_v[pl.ds(r * HWORDS + b * L, L)]
            li = li + lab_v[pl.ds(r * HWORDS + b * L, L)]
            pi = pi + prd_v[pl.ds(r * HWORDS + b * L, L)]
        cnt_v[pl.ds(b * L, L)] = ci
        lab_v[pl.ds(b * L, L)] = li
        prd_v[pl.ds(b * L, L)] = pi

    pltpu.sync_copy(cnt_v.at[pl.ds(0, HWORDS)], cnt_out.at[wid])
    pltpu.sync_copy(lab_v.at[pl.ds(0, HWORDS)], lab_out.at[wid])
    pltpu.sync_copy(prd_v.at[pl.ds(0, HWORDS)], prd_out.at[wid])


TC_RB = 512
TC_G = (N_TC // TILE) // TC_RB
_LG_ROW0 = (N_SC * 2) // TILE // (2 * TC_RB)
_LB_ROW0 = (N_SC // TILE) // TC_RB


def _tc_body(lg_ref, lb_ref, out_ref, acc_ref):
    g = pl.program_id(0)
    par = (jax.lax.broadcasted_iota(jnp.int32, (8, 128), 0) % 2) == 0
    idx_a = jax.lax.broadcasted_iota(jnp.int32, (8, 128), 0) // 2
    idx_b = idx_a + 4
    zero8 = jnp.zeros((8, 128), jnp.float32)

    @pl.when(g == 0)
    def _():
        acc_ref[...] = jnp.zeros_like(acc_ref)

    def slab_pair(j, accs):
        xa = lg_ref[pl.ds(16 * j, 8), :]
        xb = lg_ref[pl.ds(16 * j + 8, 8), :]
        lbf = lb_ref[pl.ds(8 * j, 8), :].astype(jnp.float32)
        new = []
        wla = 1.0 + 2048.0 * jnp.take_along_axis(lbf, idx_a, axis=0)
        wlb = 1.0 + 2048.0 * jnp.take_along_axis(lbf, idx_b, axis=0)
        pa = 1.0 / (1.0 + jnp.exp(xa - pltpu.roll(xa, 7, 0)))
        pb = 1.0 / (1.0 + jnp.exp(xb - pltpu.roll(xb, 7, 0)))
        bina = (pa * 10.0).astype(jnp.int32)
        binb = (pb * 10.0).astype(jnp.int32)
        for b in range(HBINS):
            ma = (bina == b) & par
            mb = (binb == b) & par
            cacc = accs[2 * b] + jnp.where(ma, wla, zero8) \
                + jnp.where(mb, wlb, zero8)
            pacc = accs[2 * b + 1] + jnp.where(ma, pa, zero8) \
                + jnp.where(mb, pb, zero8)
            new.append(cacc)
            new.append(pacc)
        return tuple(new)

    accs0 = tuple(acc_ref[i] for i in range(2 * HBINS))
    accs1 = lax.fori_loop(0, TC_RB // 8, slab_pair, accs0)
    for i in range(2 * HBINS):
        acc_ref[i] = accs1[i]

    @pl.when(g == TC_G - 1)
    def _():
        out_ref[...] = acc_ref[...]


_tc_hist = pl.pallas_call(
    _tc_body,
    grid=(TC_G,),
    in_specs=[
        pl.BlockSpec((2 * TC_RB, 128), lambda g: (_LG_ROW0 + g, 0)),
        pl.BlockSpec((TC_RB, 128), lambda g: (_LB_ROW0 + g, 0)),
    ],
    out_specs=pl.BlockSpec((2 * HBINS, 8, 128), lambda g: (0, 0, 0)),
    out_shape=jax.ShapeDtypeStruct((2 * HBINS, 8, 128), jnp.float32),
    scratch_shapes=[pltpu.VMEM((2 * HBINS, 8, 128), jnp.float32)],
)


def kernel(logits, labels):
    lg_flat = (logits.reshape(N_TOTAL // TILE, TILE, 2)
               .transpose(0, 2, 1).reshape(-1))
    cnt, lab, prd = _ece_hist(lg_flat, labels)
    tc = _tc_hist(lg_flat.reshape(N_TOTAL * 2 // TILE, TILE),
                  labels.reshape(N_TOTAL // TILE, TILE))
    combo = tc[0::2].astype(jnp.int32)
    cnt_tc = (combo % 2048).sum(axis=(1, 2)).astype(jnp.float32)
    lab_tc = (combo // 2048).sum(axis=(1, 2)).astype(jnp.float32)
    prd_tc = tc[1::2].sum(axis=(1, 2))
    sizes = cnt.reshape(NW, HBINS, L).sum(axis=(0, 2)).astype(jnp.float32)
    lab_s = lab.reshape(NW, HBINS, L).sum(axis=(0, 2)).astype(jnp.float32)
    prd_s = prd.reshape(NW, HBINS, L).sum(axis=(0, 2))
    sizes = (sizes + cnt_tc)[:N_BINS_OUT]
    lab_s = (lab_s + lab_tc)[:N_BINS_OUT]
    prd_s = (prd_s + prd_tc)[:N_BINS_OUT]
    nonempty = sizes > 0
    safe = jnp.where(nonempty, sizes, 1.0)
    accs = jnp.where(nonempty, lab_s / safe, 0.0)
    confs = jnp.where(nonempty, prd_s / safe, 0.0)
    return jnp.sum(sizes / jnp.sum(sizes) * jnp.abs(accs - confs))


# speedup vs baseline: 2.4857x; 1.0943x over previous
import functools

import jax
import jax.numpy as jnp
from jax import lax
from jax.experimental import pallas as pl
from jax.experimental.pallas import tpu as pltpu
from jax.experimental.pallas import tpu_sc as plsc

N_TOTAL = 2097152
N_SC = 1572864
N_TC = N_TOTAL - N_SC
N_BINS_OUT = 10
NC = 2
NS = 16
L = 16
NW = NC * NS
PER_W = N_SC // NW
BLK = 8192
NBLK = PER_W // BLK
TILE = 128
NTILE = BLK // TILE
HBINS = 11
HWORDS = HBINS * L
NREP = TILE // L
HTOT = NREP * HWORDS

_mesh = plsc.VectorSubcoreMesh(core_axis_name="c", subcore_axis_name="s")


@functools.partial(
    pl.kernel,
    mesh=_mesh,
    out_type=jax.ShapeDtypeStruct((NW * 3 * HWORDS,), jnp.float32),
    scratch_types=[
        pltpu.VMEM((2 * BLK,), jnp.float32),
        pltpu.VMEM((2 * BLK,), jnp.float32),
        pltpu.VMEM((BLK,), jnp.int32),
        pltpu.VMEM((BLK,), jnp.int32),
        pltpu.VMEM((HTOT,), jnp.float32),
        pltpu.VMEM((HTOT,), jnp.float32),
        pltpu.VMEM((HTOT,), jnp.float32),
        pltpu.SemaphoreType.DMA,
        pltpu.SemaphoreType.DMA,
    ],
    compiler_params=pltpu.CompilerParams(
        needs_layout_passes=False, use_tc_tiling_on_sc=False),
)
def _ece_hist(lg_hbm, lb_hbm, hist_out,
              lg_a, lg_b, lb_a, lb_b, cnt_v, lab_v, prd_v, sem_a, sem_b):
    wid = lax.axis_index("s") * NC + lax.axis_index("c")

    lane = lax.iota(jnp.int32, L)
    ones_f = jnp.ones((L,), jnp.float32)
    z_f = jnp.zeros((L,), jnp.float32)

    for b in range(NREP * HBINS):
        cnt_v[pl.ds(b * L, L)] = z_f
        lab_v[pl.ds(b * L, L)] = z_f
        prd_v[pl.ds(b * L, L)] = z_f

    elem0 = wid * PER_W

    def start_blk(blk, lgbuf, lbbuf, sem):
        off = elem0 + blk * BLK
        pltpu.async_copy(lg_hbm.at[pl.ds(off * 2, 2 * BLK)], lgbuf, sem)
        pltpu.async_copy(lb_hbm.at[pl.ds(off, BLK)], lbbuf, sem)

    def wait_blk(lgbuf, lbbuf, sem):
        pltpu.make_async_copy(lg_hbm.at[pl.ds(0, 2 * BLK)], lgbuf, sem).wait()
        pltpu.make_async_copy(lb_hbm.at[pl.ds(0, BLK)], lbbuf, sem).wait()

    def compute(lg_v, lb_v):
        def body(t):
            base = t * (2 * TILE)
            lbase = t * TILE
            for i in range(TILE // L):
                l0 = lg_v[pl.ds(base + i * L, L)]
                l1 = lg_v[pl.ds(base + TILE + i * L, L)]
                lbf = lb_v[pl.ds(lbase + i * L, L)].astype(jnp.float32)
                e = jnp.exp(l0 - l1)
                p = 1.0 / (1.0 + e)
                bin_ = (p * 10.0).astype(jnp.int32)
                addr = bin_ * L + lane
                rep = pl.ds(i * HWORDS, HWORDS)
                plsc.addupdate_scatter(cnt_v.at[rep], [addr], ones_f)
                plsc.addupdate_scatter(lab_v.at[rep], [addr], lbf)
                plsc.addupdate_scatter(prd_v.at[rep], [addr], p)

        plsc.parallel_loop(0, NTILE)(body)

    start_blk(0, lg_a, lb_a, sem_a)

    def super_body(k, c):
        blk_a = 2 * k
        wait_blk(lg_a, lb_a, sem_a)
        start_blk(blk_a + 1, lg_b, lb_b, sem_b)
        compute(lg_a, lb_a)
        wait_blk(lg_b, lb_b, sem_b)

        @pl.when(k < NBLK // 2 - 1)
        def _():
            start_blk(blk_a + 2, lg_a, lb_a, sem_a)

        compute(lg_b, lb_b)
        return c

    lax.fori_loop(0, NBLK // 2, super_body, 0)

    for b in range(HBINS):
        ci = cnt_v[pl.ds(b * L, L)]
        li = lab_v[pl.ds(b * L, L)]
        pi = prd_v[pl.ds(b * L, L)]
        for r in range(1, NREP):
            ci = ci + cnt_v[pl.ds(r * HWORDS + b * L, L)]
            li = li + lab_v[pl.ds(r * HWORDS + b * L, L)]
            pi = pi + prd_v[pl.ds(r * HWORDS + b * L, L)]
        cnt_v[pl.ds(b * L, L)] = ci
        lab_v[pl.ds(b * L, L)] = li
        prd_v[pl.ds(b * L, L)] = pi

    row = wid * (3 * HWORDS)
    pltpu.sync_copy(cnt_v.at[pl.ds(0, HWORDS)], hist_out.at[pl.ds(row, HWORDS)])
    pltpu.sync_copy(lab_v.at[pl.ds(0, HWORDS)],
                    hist_out.at[pl.ds(row + HWORDS, HWORDS)])
    pltpu.sync_copy(prd_v.at[pl.ds(0, HWORDS)],
                    hist_out.at[pl.ds(row + 2 * HWORDS, HWORDS)])


TC_RB = 512
TC_G = (N_TC // TILE) // TC_RB
_LG_ROW0 = (N_SC * 2) // TILE // (2 * TC_RB)
_LB_ROW0 = (N_SC // TILE) // TC_RB


def _tc_body(lg_ref, lb_ref, out_ref, acc_ref):
    g = pl.program_id(0)
    par = (jax.lax.broadcasted_iota(jnp.int32, (8, 128), 0) % 2) == 0
    idx_a = jax.lax.broadcasted_iota(jnp.int32, (8, 128), 0) // 2
    idx_b = idx_a + 4
    zero8 = jnp.zeros((8, 128), jnp.float32)

    @pl.when(g == 0)
    def _():
        acc_ref[...] = jnp.zeros_like(acc_ref)

    def slab_pair(j, accs):
        xa = lg_ref[pl.ds(16 * j, 8), :]
        xb = lg_ref[pl.ds(16 * j + 8, 8), :]
        lbf = lb_ref[pl.ds(8 * j, 8), :].astype(jnp.float32)
        new = []
        wla = 1.0 + 2048.0 * jnp.take_along_axis(lbf, idx_a, axis=0)
        wlb = 1.0 + 2048.0 * jnp.take_along_axis(lbf, idx_b, axis=0)
        pa = 1.0 / (1.0 + jnp.exp(xa - pltpu.roll(xa, 7, 0)))
        pb = 1.0 / (1.0 + jnp.exp(xb - pltpu.roll(xb, 7, 0)))
        bina = (pa * 10.0).astype(jnp.int32)
        binb = (pb * 10.0).astype(jnp.int32)
        for b in range(HBINS):
            ma = (bina == b) & par
            mb = (binb == b) & par
            cacc = accs[2 * b] + jnp.where(ma, wla, zero8) \
                + jnp.where(mb, wlb, zero8)
            pacc = accs[2 * b + 1] + jnp.where(ma, pa, zero8) \
                + jnp.where(mb, pb, zero8)
            new.append(cacc)
            new.append(pacc)
        return tuple(new)

    accs0 = tuple(acc_ref[i] for i in range(2 * HBINS))
    accs1 = lax.fori_loop(0, TC_RB // 8, slab_pair, accs0)
    for i in range(2 * HBINS):
        acc_ref[i] = accs1[i]

    @pl.when(g == TC_G - 1)
    def _():
        out_ref[...] = acc_ref[...]


_tc_hist = pl.pallas_call(
    _tc_body,
    grid=(TC_G,),
    in_specs=[
        pl.BlockSpec((2 * TC_RB, 128), lambda g: (_LG_ROW0 + g, 0)),
        pl.BlockSpec((TC_RB, 128), lambda g: (_LB_ROW0 + g, 0)),
    ],
    out_specs=pl.BlockSpec((2 * HBINS, 8, 128), lambda g: (0, 0, 0)),
    out_shape=jax.ShapeDtypeStruct((2 * HBINS, 8, 128), jnp.float32),
    scratch_shapes=[pltpu.VMEM((2 * HBINS, 8, 128), jnp.float32)],
)


def kernel(logits, labels):
    lg_flat = (logits.reshape(N_TOTAL // TILE, TILE, 2)
               .transpose(0, 2, 1).reshape(-1))
    hist = _ece_hist(lg_flat, labels)
    tc = _tc_hist(lg_flat.reshape(N_TOTAL * 2 // TILE, TILE),
                  labels.reshape(N_TOTAL // TILE, TILE))
    combo = tc[0::2].astype(jnp.int32)
    cnt_tc = (combo % 2048).sum(axis=(1, 2)).astype(jnp.float32)
    lab_tc = (combo // 2048).sum(axis=(1, 2)).astype(jnp.float32)
    prd_tc = tc[1::2].sum(axis=(1, 2))
    hsc = hist.reshape(NW, 3, HBINS, L).sum(axis=(0, 3))
    sizes = (hsc[0] + cnt_tc)[:N_BINS_OUT]
    lab_s = (hsc[1] + lab_tc)[:N_BINS_OUT]
    prd_s = (hsc[2] + prd_tc)[:N_BINS_OUT]
    nonempty = sizes > 0
    safe = jnp.where(nonempty, sizes, 1.0)
    accs = jnp.where(nonempty, lab_s / safe, 0.0)
    confs = jnp.where(nonempty, prd_s / safe, 0.0)
    return jnp.sum(sizes / jnp.sum(sizes) * jnp.abs(accs - confs))
